# Initial kernel scaffold; baseline (speedup 1.0000x reference)
import jax
import jax.numpy as jnp
from jax.experimental import pallas as pl


def kernel(gate_logits, expert_logits, node_masks, edge_masks, loss_reg, loss_sem, loss_str, y, epoch):
    raise NotImplementedError("write your pallas kernel here")


# speedup vs baseline: 3.4281x; 3.4281x over previous
import jax
import jax.numpy as jnp
from jax import lax
from jax.experimental import pallas as pl
from jax.experimental.pallas import tpu as pltpu

_E = 8
_B = 4096
_C = 2
_NN = 100000
_NE = 1600000
_TRAIN_AFTER = 10
_ALPHA = 1.5
_W_CE, _W_REG, _W_SEM, _W_STR, _W_DIV, _W_LOAD = 1.0, 0.5, 0.5, 0.5, 0.1, 0.01

_CHUNK = 160000
_NSTEP = _NE // _CHUNK


def _entmax_bisect_t(X):
    d = X.shape[0]
    Xs = X * (_ALPHA - 1.0)
    max_val = jnp.max(Xs, axis=0, keepdims=True)

    def _p(z):
        zc = jnp.maximum(z, 0.0)
        return zc * zc

    tau_lo = max_val - 1.0
    tau_hi = max_val - (1.0 / d) ** (_ALPHA - 1.0)
    f_lo = jnp.sum(_p(Xs - tau_lo), axis=0, keepdims=True) - 1.0
    dm = tau_hi - tau_lo
    p_m = _p(Xs - tau_lo)
    for _ in range(50):
        dm = dm / 2.0
        tau_m = tau_lo + dm
        p_m = _p(Xs - tau_m)
        f_m = jnp.sum(p_m, axis=0, keepdims=True) - 1.0
        mask = (f_m * f_lo) >= 0
        tau_lo = jnp.where(mask, tau_m, tau_lo)
    return p_m / jnp.sum(p_m, axis=0, keepdims=True)


def _offdiag_mean_from_gram(G):
    K = G.shape[0]
    eye = (lax.broadcasted_iota(jnp.int32, (K, K), 0)
           == lax.broadcasted_iota(jnp.int32, (K, K), 1))
    eyef = eye.astype(jnp.float32)
    diag_row = jnp.sum(G * eyef, axis=0, keepdims=True)
    ninv_row = 1.0 / jnp.maximum(jnp.sqrt(diag_row), 1e-12)
    ninv_col = jnp.sum(eyef * ninv_row, axis=1, keepdims=True)
    S = G * ninv_col * ninv_row
    full = jnp.sum(S)
    diag = jnp.sum(S * eyef)
    return (full - diag) / (K * (K - 1))


def _body(flag_ref, gate_ref, el0_ref, el1_ref, node_ref, y_ref,
          reg_ref, sem_ref, str_ref, edge_ref,
          agg_ref, total_ref, acc_ref, part_ref):
    i = pl.program_id(0)

    @pl.when(i == 0)
    def _init():
        acc_ref[:, :] = jnp.zeros((_E, _E), jnp.float32)

    x = edge_ref[:, :]
    acc_ref[:, :] += lax.dot_general(
        x, x, (((1,), (1,)), ((), ())), preferred_element_type=jnp.float32)

    @pl.when(i == 0)
    def _small():
        nm = node_ref[:, :]
        Gn = lax.dot_general(nm, nm, (((1,), (1,)), ((), ())),
                             preferred_element_type=jnp.float32)
        off_node = _offdiag_mean_from_gram(Gn)

        gate = gate_ref[:, :]
        uniform = jnp.full((_E, _B), 1.0 / _E, jnp.float32)
        gw0 = jnp.where(flag_ref[:, :] > 0.0, uniform, gate)
        gw = _entmax_bisect_t(gw0)

        agg0 = jnp.sum(el0_ref[:, :] * gw, axis=0, keepdims=True)
        agg1 = jnp.sum(el1_ref[:, :] * gw, axis=0, keepdims=True)
        agg_ref[0:1, :] = agg0
        agg_ref[1:2, :] = agg1

        yf = y_ref[:, :].astype(jnp.float32)
        c1 = jnp.sum(yf)
        c0 = jnp.float32(_B) - c1
        c0 = jnp.where(c0 == 0.0, 1.0, c0)
        c1 = jnp.where(c1 == 0.0, 1.0, c1)
        w0 = 1.0 / c0
        w1 = 1.0 / c1
        wsum = w0 + w1
        w0 = w0 / wsum
        w1 = w1 / wsum
        m = jnp.maximum(agg0, agg1)
        lse = m + jnp.log(jnp.exp(agg0 - m) + jnp.exp(agg1 - m))
        logp0 = agg0 - lse
        logp1 = agg1 - lse
        is0 = y_ref[:, :] == 0
        nll = -jnp.where(is0, logp0, logp1)
        wi = jnp.where(is0, w0, w1)
        ce = jnp.sum(wi * nll) / jnp.sum(wi)

        w_first = gw[:, 0:1]
        reg = jnp.sum(w_first * reg_ref[:, :])
        sem = jnp.sum(w_first * sem_ref[:, :])
        strv = jnp.sum(w_first * str_ref[:, :])

        avg = jnp.sum(gw, axis=1, keepdims=True) / jnp.float32(_B)
        u = 1.0 / _E
        load = jnp.sum(u * (jnp.log(jnp.full((_E, 1), u, jnp.float32))
                            - jnp.log(avg + 1e-8))) / _E

        part_ref[0] = (_W_CE * ce + _W_REG * reg + _W_SEM * sem
                       + _W_STR * strv + _W_LOAD * load)
        part_ref[1] = off_node

    @pl.when(i == _NSTEP - 1)
    def _final():
        off_edge = _offdiag_mean_from_gram(acc_ref[:, :])
        div = (part_ref[1] + off_edge) / 2.0
        total_ref[0:1, 0:1] = jnp.reshape(part_ref[0] + _W_DIV * div, (1, 1))


def kernel(gate_logits, expert_logits, node_masks, edge_masks,
           loss_reg, loss_sem, loss_str, y, epoch):
    flag = (jnp.asarray(epoch, jnp.int32) < _TRAIN_AFTER).astype(
        jnp.float32).reshape(1, 1)
    gate_t = gate_logits.T
    el0 = expert_logits[:, :, 0]
    el1 = expert_logits[:, :, 1]
    y2 = y.reshape(1, _B)
    reg2 = loss_reg.reshape(_E, 1)
    sem2 = loss_sem.reshape(_E, 1)
    str2 = loss_str.reshape(_E, 1)

    agg_t, total = pl.pallas_call(
        _body,
        grid=(_NSTEP,),
        in_specs=[
            pl.BlockSpec((1, 1), lambda i: (0, 0)),
            pl.BlockSpec((_E, _B), lambda i: (0, 0)),
            pl.BlockSpec((_E, _B), lambda i: (0, 0)),
            pl.BlockSpec((_E, _B), lambda i: (0, 0)),
            pl.BlockSpec((_E, _NN), lambda i: (0, 0)),
            pl.BlockSpec((1, _B), lambda i: (0, 0)),
            pl.BlockSpec((_E, 1), lambda i: (0, 0)),
            pl.BlockSpec((_E, 1), lambda i: (0, 0)),
            pl.BlockSpec((_E, 1), lambda i: (0, 0)),
            pl.BlockSpec((_E, _CHUNK), lambda i: (0, i)),
        ],
        out_specs=[
            pl.BlockSpec((_C, _B), lambda i: (0, 0)),
            pl.BlockSpec((1, 1), lambda i: (0, 0)),
        ],
        out_shape=[
            jax.ShapeDtypeStruct((_C, _B), jnp.float32),
            jax.ShapeDtypeStruct((1, 1), jnp.float32),
        ],
        scratch_shapes=[
            pltpu.VMEM((_E, _E), jnp.float32),
            pltpu.SMEM((2,), jnp.float32),
        ],
    )(flag, gate_t, el0, el1, node_masks, y2, reg2, sem2, str2, edge_masks)

    return agg_t.T, total.reshape(())
